# baseline (speedup 1.0000x reference)
import functools

import jax
import jax.numpy as jnp
from jax import lax
from jax.experimental import pallas as pl
from jax.experimental.pallas import tpu as pltpu
from jax.experimental.pallas import tpu_sc as plsc

D_MODEL = 64
SCALE = 8.0
LANES = 16

NC = 2
NS = 16
NW = NC * NS

B_TOTAL = 1024 * 200
B_PER_W = B_TOTAL // NW
CHUNK = 128
N_CHUNKS = B_PER_W // CHUNK
NBUF = 2

_mesh = plsc.VectorSubcoreMesh(core_axis_name="c", subcore_axis_name="s")


@functools.partial(
    pl.kernel,
    mesh=_mesh,
    compiler_params=pltpu.CompilerParams(use_tc_tiling_on_sc=False),
    out_type=jax.ShapeDtypeStruct((B_TOTAL, D_MODEL), jnp.float32),
    scratch_types=[
        pltpu.VMEM((N_CHUNKS, CHUNK), jnp.int32),
        pltpu.VMEM((NBUF, CHUNK, D_MODEL), jnp.float32),
        pltpu.VMEM((NBUF, CHUNK, D_MODEL), jnp.float32),
        pltpu.SemaphoreType.DMA,
        pltpu.SemaphoreType.DMA,
        pltpu.SemaphoreType.DMA,
        pltpu.SemaphoreType.DMA,
    ],
)
def _emb_lookup(x_hbm, table_hbm, out_hbm, idx_v, gbuf, sbuf,
                gsem0, gsem1, ssem0, ssem1):
    gsems = (gsem0, gsem1)
    ssems = (ssem0, ssem1)
    wid = lax.axis_index("s") * NC + lax.axis_index("c")
    base = wid * B_PER_W

    pltpu.sync_copy(x_hbm.at[wid], idx_v)

    def gather_copy(g, slot):
        return pltpu.make_async_copy(
            table_hbm.at[idx_v.at[g]], gbuf.at[slot], gsems[slot])

    def scatter_copy(g, slot):
        return pltpu.make_async_copy(
            sbuf.at[slot], out_hbm.at[pl.ds(base + g * CHUNK, CHUNK)],
            ssems[slot])

    def scale_chunk(slot):
        def row(r, carry):
            for c in range(D_MODEL // LANES):
                sl = pl.ds(c * LANES, LANES)
                sbuf[slot, r, sl] = gbuf[slot, r, sl] * SCALE
            return carry
        lax.fori_loop(0, CHUNK, row, 0, unroll=4)

    for b in range(NBUF):
        gather_copy(b, b).start()

    for b in range(NBUF):
        gather_copy(b, b).wait()
        scale_chunk(b)
        gather_copy(b + NBUF, b).start()
        scatter_copy(b, b).start()

    def outer(j, carry):
        for b in range(NBUF):
            g = j * NBUF + b
            gather_copy(g, b).wait()
            scatter_copy(g - NBUF, b).wait()
            scale_chunk(b)
            gather_copy(g + NBUF, b).start()
            scatter_copy(g, b).start()
        return carry
    lax.fori_loop(1, N_CHUNKS // NBUF - 1, outer, 0)

    for b in range(NBUF):
        g = N_CHUNKS - NBUF + b
        gather_copy(g, b).wait()
        scatter_copy(g - NBUF, b).wait()
        scale_chunk(b)
        scatter_copy(g, b).start()

    for b in range(NBUF):
        scatter_copy(N_CHUNKS - NBUF + b, b).wait()


def kernel(x, emb_table):
    x3 = x.reshape(NW, N_CHUNKS, CHUNK).astype(jnp.int32)
    out = _emb_lookup(x3, emb_table)
    return out.reshape(x.shape[0], x.shape[1], D_MODEL)


# speedup vs baseline: 1.0785x; 1.0785x over previous
import functools

import jax
import jax.numpy as jnp
from jax import lax
from jax.experimental import pallas as pl
from jax.experimental.pallas import tpu as pltpu
from jax.experimental.pallas import tpu_sc as plsc

D_MODEL = 64
SCALE = 8.0
LANES = 16

NC = 2
NS = 16
NW = NC * NS

B_TOTAL = 1024 * 200
B_PER_W = B_TOTAL // NW
CHUNK = 128
N_CHUNKS = B_PER_W // CHUNK
NBUF = 2

_mesh = plsc.VectorSubcoreMesh(core_axis_name="c", subcore_axis_name="s")


@functools.partial(
    pl.kernel,
    mesh=_mesh,
    compiler_params=pltpu.CompilerParams(use_tc_tiling_on_sc=True),
    out_type=jax.ShapeDtypeStruct((B_TOTAL, D_MODEL), jnp.float32),
    scratch_types=[
        pltpu.VMEM((N_CHUNKS, CHUNK), jnp.int32),
        pltpu.VMEM((N_CHUNKS, CHUNK), jnp.int32),
        pltpu.VMEM((NBUF, CHUNK, 2 * D_MODEL), jnp.float32),
        pltpu.VMEM((NBUF, CHUNK, D_MODEL), jnp.float32),
        pltpu.SemaphoreType.DMA,
        pltpu.SemaphoreType.DMA,
        pltpu.SemaphoreType.DMA,
        pltpu.SemaphoreType.DMA,
    ],
)
def _emb_lookup(xp_hbm, xq_hbm, tpair_hbm, out_hbm, pidx_v, poff_v,
                gbuf, sbuf, gsem0, gsem1, ssem0, ssem1):
    gsems = (gsem0, gsem1)
    ssems = (ssem0, ssem1)
    wid = lax.axis_index("s") * NC + lax.axis_index("c")
    base = wid * B_PER_W

    pltpu.sync_copy(xp_hbm.at[wid], pidx_v)
    pltpu.sync_copy(xq_hbm.at[wid], poff_v)

    def gather_copy(g, slot):
        return pltpu.make_async_copy(
            tpair_hbm.at[pidx_v.at[g]], gbuf.at[slot], gsems[slot])

    def scatter_copy(g, slot):
        return pltpu.make_async_copy(
            sbuf.at[slot], out_hbm.at[pl.ds(base + g * CHUNK, CHUNK)],
            ssems[slot])

    def scale_chunk(g, slot):
        def grp(k, carry):
            offs = poff_v[g, pl.ds(k * LANES, LANES)]
            for l in range(LANES):
                off = offs[l]
                r = k * LANES + l
                for c in range(D_MODEL // LANES):
                    sbuf[slot, r, pl.ds(c * LANES, LANES)] = (
                        gbuf[slot, r, pl.ds(off + c * LANES, LANES)] * SCALE)
            return carry
        lax.fori_loop(0, CHUNK // LANES, grp, 0)

    for b in range(NBUF):
        gather_copy(b, b).start()

    for b in range(NBUF):
        gather_copy(b, b).wait()
        scale_chunk(b, b)
        gather_copy(b + NBUF, b).start()
        scatter_copy(b, b).start()

    def outer(j, carry):
        for b in range(NBUF):
            g = j * NBUF + b
            gather_copy(g, b).wait()
            scatter_copy(g - NBUF, b).wait()
            scale_chunk(g, b)
            gather_copy(g + NBUF, b).start()
            scatter_copy(g, b).start()
        return carry
    lax.fori_loop(1, N_CHUNKS // NBUF - 1, outer, 0)

    for b in range(NBUF):
        g = N_CHUNKS - NBUF + b
        gather_copy(g, b).wait()
        scatter_copy(g - NBUF, b).wait()
        scale_chunk(g, b)
        scatter_copy(g, b).start()

    for b in range(NBUF):
        scatter_copy(N_CHUNKS - NBUF + b, b).wait()


def kernel(x, emb_table):
    xi = x.astype(jnp.int32)
    xp = (xi >> 1).reshape(NW, N_CHUNKS, CHUNK)
    xq = ((xi & 1) * D_MODEL).reshape(NW, N_CHUNKS, CHUNK)
    tpair = emb_table.reshape(500000, 2 * D_MODEL)
    out = _emb_lookup(xp, xq, tpair)
    return out.reshape(x.shape[0], x.shape[1], D_MODEL)


# speedup vs baseline: 1.3255x; 1.2291x over previous
import functools

import jax
import jax.numpy as jnp
from jax import lax
from jax.experimental import pallas as pl
from jax.experimental.pallas import tpu as pltpu
from jax.experimental.pallas import tpu_sc as plsc

D_MODEL = 64
SCALE = 8.0
LANES = 16

NC = 2
NS = 16
NW = NC * NS

B_TOTAL = 1024 * 200
B_PER_W = B_TOTAL // NW
CHUNK = 128
N_CHUNKS = B_PER_W // CHUNK
NBUF = 2

_mesh = plsc.VectorSubcoreMesh(core_axis_name="c", subcore_axis_name="s")


@functools.partial(
    pl.kernel,
    mesh=_mesh,
    compiler_params=pltpu.CompilerParams(use_tc_tiling_on_sc=True),
    out_type=jax.ShapeDtypeStruct((B_TOTAL, D_MODEL), jnp.float32),
    scratch_types=[
        pltpu.VMEM((N_CHUNKS, CHUNK), jnp.int32),
        pltpu.VMEM((N_CHUNKS, CHUNK), jnp.int32),
        pltpu.VMEM((NBUF, CHUNK, 2 * D_MODEL), jnp.float32),
        pltpu.VMEM((NBUF, CHUNK, D_MODEL), jnp.float32),
        pltpu.SemaphoreType.DMA,
        pltpu.SemaphoreType.DMA,
        pltpu.SemaphoreType.DMA,
        pltpu.SemaphoreType.DMA,
    ],
)
def _emb_lookup(xp_hbm, xq_hbm, tpair_hbm, out_hbm, pidx_v, poff_v,
                gbuf, sbuf, gsem0, gsem1, ssem0, ssem1):
    gsems = (gsem0, gsem1)
    ssems = (ssem0, ssem1)
    wid = lax.axis_index("s") * NC + lax.axis_index("c")
    base = wid * B_PER_W

    pltpu.sync_copy(xp_hbm.at[wid], pidx_v)
    pltpu.sync_copy(xq_hbm.at[wid], poff_v)

    def gather_copy(g, slot):
        return pltpu.make_async_copy(
            tpair_hbm.at[pidx_v.at[g]], gbuf.at[slot], gsems[slot])

    def scatter_copy(g, slot):
        return pltpu.make_async_copy(
            sbuf.at[slot], out_hbm.at[pl.ds(base + g * CHUNK, CHUNK)],
            ssems[slot])

    def scale_chunk(g, slot):
        def grp(k, carry):
            offs = poff_v[g, pl.ds(k * LANES, LANES)]
            for l in range(LANES):
                off = offs[l]
                r = k * LANES + l
                for c in range(D_MODEL // LANES):
                    sbuf[slot, r, pl.ds(c * LANES, LANES)] = (
                        gbuf[slot, r, pl.ds(off + c * LANES, LANES)] * SCALE)
            return carry
        lax.fori_loop(0, CHUNK // LANES, grp, 0)

    for b in range(NBUF):
        gather_copy(b, b).start()

    for b in range(NBUF):
        gather_copy(b, b).wait()
        scale_chunk(b, b)
        gather_copy(b + NBUF, b).start()
        scatter_copy(b, b).start()

    def outer(j, carry):
        for b in range(NBUF):
            g = j * NBUF + b
            gather_copy(g, b).wait()
            scatter_copy(g - NBUF, b).wait()
            scale_chunk(g, b)
            gather_copy(g + NBUF, b).start()
            scatter_copy(g, b).start()
        return carry
    lax.fori_loop(1, N_CHUNKS // NBUF - 1, outer, 0)

    for b in range(NBUF):
        g = N_CHUNKS - NBUF + b
        gather_copy(g, b).wait()
        scatter_copy(g - NBUF, b).wait()
        scale_chunk(g, b)
        scatter_copy(g, b).start()

    for b in range(NBUF):
        scatter_copy(N_CHUNKS - NBUF + b, b).wait()


_TBLK = 2048


def _transpose_body(xt_ref, out_ref):
    lo = xt_ref[:, : _TBLK // 2].T
    hi = xt_ref[:, _TBLK // 2 :].T
    out_ref[...] = lax.concatenate([lo, hi], 1)


_transpose_table = pl.pallas_call(
    _transpose_body,
    grid=(1000000 // _TBLK,),
    in_specs=[pl.BlockSpec((D_MODEL, _TBLK), lambda i: (0, i))],
    out_specs=pl.BlockSpec((_TBLK // 2, 2 * D_MODEL), lambda i: (i, 0)),
    out_shape=jax.ShapeDtypeStruct((500000, 2 * D_MODEL), jnp.float32),
)


def kernel(x, emb_table):
    xi = x.astype(jnp.int32)
    xp = (((xi >> 11) << 10) | (xi & 1023)).reshape(NW, N_CHUNKS, CHUNK)
    xq = (((xi >> 10) & 1) * D_MODEL).reshape(NW, N_CHUNKS, CHUNK)
    tpair = _transpose_table(emb_table.T)
    out = _emb_lookup(xp, xq, tpair)
    return out.reshape(x.shape[0], x.shape[1], D_MODEL)


# speedup vs baseline: 1.3270x; 1.0011x over previous
import functools

import jax
import jax.numpy as jnp
from jax import lax
from jax.experimental import pallas as pl
from jax.experimental.pallas import tpu as pltpu
from jax.experimental.pallas import tpu_sc as plsc

D_MODEL = 64
SCALE = 8.0
LANES = 16

NC = 2
NS = 16
NW = NC * NS

B_TOTAL = 1024 * 200
B_PER_W = B_TOTAL // NW
CHUNK = 128
N_CHUNKS = B_PER_W // CHUNK
NBUF = 2

_mesh = plsc.VectorSubcoreMesh(core_axis_name="c", subcore_axis_name="s")


@functools.partial(
    pl.kernel,
    mesh=_mesh,
    compiler_params=pltpu.CompilerParams(use_tc_tiling_on_sc=True),
    out_type=jax.ShapeDtypeStruct((B_TOTAL, D_MODEL), jnp.float32),
    scratch_types=[
        pltpu.VMEM((N_CHUNKS, CHUNK), jnp.int32),
        pltpu.VMEM((N_CHUNKS, CHUNK), jnp.int32),
        pltpu.VMEM((NBUF, CHUNK, 2 * D_MODEL), jnp.float32),
        pltpu.VMEM((NBUF, CHUNK, D_MODEL), jnp.float32),
        pltpu.SemaphoreType.DMA,
        pltpu.SemaphoreType.DMA,
        pltpu.SemaphoreType.DMA,
        pltpu.SemaphoreType.DMA,
    ],
)
def _emb_lookup(xp_hbm, xq_hbm, tpair_hbm, out_hbm, pidx_v, poff_v,
                gbuf, sbuf, gsem0, gsem1, ssem0, ssem1):
    gsems = (gsem0, gsem1)
    ssems = (ssem0, ssem1)
    wid = lax.axis_index("s") * NC + lax.axis_index("c")
    base = wid * B_PER_W

    pltpu.sync_copy(xp_hbm.at[wid], pidx_v)
    pltpu.sync_copy(xq_hbm.at[wid], poff_v)

    def gather_copy(g, slot):
        return pltpu.make_async_copy(
            tpair_hbm.at[pidx_v.at[g]], gbuf.at[slot], gsems[slot])

    def scatter_copy(g, slot):
        return pltpu.make_async_copy(
            sbuf.at[slot], out_hbm.at[pl.ds(base + g * CHUNK, CHUNK)],
            ssems[slot])

    def scale_chunk(g, slot):
        def grp(k, carry):
            offs = poff_v[g, pl.ds(k * LANES, LANES)]
            for l in range(LANES):
                off = offs[l]
                r = k * LANES + l
                for c in range(D_MODEL // LANES):
                    sbuf[slot, r, pl.ds(c * LANES, LANES)] = (
                        gbuf[slot, r, pl.ds(off + c * LANES, LANES)] * SCALE)
            return carry
        lax.fori_loop(0, CHUNK // LANES, grp, 0)

    for b in range(NBUF):
        gather_copy(b, b).start()

    for b in range(NBUF):
        gather_copy(b, b).wait()
        scale_chunk(b, b)
        gather_copy(b + NBUF, b).start()
        scatter_copy(b, b).start()

    def outer(j, carry):
        for b in range(NBUF):
            g = j * NBUF + b
            gather_copy(g, b).wait()
            scatter_copy(g - NBUF, b).wait()
            scale_chunk(g, b)
            gather_copy(g + NBUF, b).start()
            scatter_copy(g, b).start()
        return carry
    lax.fori_loop(1, N_CHUNKS // NBUF - 1, outer, 0)

    for b in range(NBUF):
        g = N_CHUNKS - NBUF + b
        gather_copy(g, b).wait()
        scatter_copy(g - NBUF, b).wait()
        scale_chunk(g, b)
        scatter_copy(g, b).start()

    for b in range(NBUF):
        scatter_copy(N_CHUNKS - NBUF + b, b).wait()


_TBLK = 2048


def _transpose_body(xt_ref, out_ref):
    lo = xt_ref[:, : _TBLK // 2].T
    hi = xt_ref[:, _TBLK // 2 :].T
    out_ref[...] = lax.concatenate([lo, hi], 1)


_NBLK = -(-1000000 // _TBLK)

_transpose_table = pl.pallas_call(
    _transpose_body,
    grid=(_NBLK,),
    in_specs=[pl.BlockSpec((D_MODEL, _TBLK), lambda i: (0, i))],
    out_specs=pl.BlockSpec((_TBLK // 2, 2 * D_MODEL), lambda i: (i, 0)),
    out_shape=jax.ShapeDtypeStruct((_NBLK * _TBLK // 2, 2 * D_MODEL),
                                   jnp.float32),
)


def kernel(x, emb_table):
    xi = x.astype(jnp.int32)
    xp = (((xi >> 11) << 10) | (xi & 1023)).reshape(NW, N_CHUNKS, CHUNK)
    xq = (((xi >> 10) & 1) * D_MODEL).reshape(NW, N_CHUNKS, CHUNK)
    tpair = _transpose_table(emb_table.T)
    out = _emb_lookup(xp, xq, tpair)
    return out.reshape(x.shape[0], x.shape[1], D_MODEL)
